# baseline (speedup 1.0000x reference)
import jax
import jax.numpy as jnp
from jax.experimental import pallas as pl

_BN = 4096


def _assign_kernel(x_ref, x2_ref, c_ref, c2_ref, out_ref):
    x = x_ref[...]
    c = c_ref[...]
    x2 = x2_ref[...]
    c2 = c2_ref[...]
    k = c.shape[0]
    h = k // 2
    dot = jnp.dot(x, c.T, preferred_element_type=jnp.float32)
    d2 = (x2 + c2) - 2.0 * dot

    da = d2[:, :h]
    db = d2[:, h:]
    iota = jax.lax.broadcasted_iota(jnp.int32, da.shape, 1)
    ma = jnp.min(da, axis=1, keepdims=True)
    ia = jnp.min(jnp.where(da == ma, iota, jnp.int32(h)), axis=1, keepdims=True)
    mb = jnp.min(db, axis=1, keepdims=True)
    ib = jnp.min(jnp.where(db == mb, iota, jnp.int32(h)), axis=1, keepdims=True)

    dist_a = jnp.sqrt(jnp.maximum(ma, 0.0))
    dist_b = jnp.sqrt(jnp.maximum(mb, 0.0))
    dist_a_r = dist_a.astype(jnp.bfloat16).astype(jnp.float32)
    pick_a = dist_a_r <= dist_b
    out_ref[...] = jnp.where(pick_a, ia, ib + jnp.int32(h))


def kernel(X, centers):
    n, d = X.shape
    k, _ = centers.shape
    x2 = jnp.sum(X * X, axis=1, keepdims=True)
    c2 = jnp.sum(centers * centers, axis=1)[None, :]
    xb = X.astype(jnp.bfloat16)
    cb = centers.astype(jnp.bfloat16)
    grid = (n // _BN,)
    out = pl.pallas_call(
        _assign_kernel,
        grid=grid,
        in_specs=[
            pl.BlockSpec((_BN, d), lambda i: (i, 0)),
            pl.BlockSpec((_BN, 1), lambda i: (i, 0)),
            pl.BlockSpec((k, d), lambda i: (0, 0)),
            pl.BlockSpec((1, k), lambda i: (0, 0)),
        ],
        out_specs=pl.BlockSpec((_BN, 1), lambda i: (i, 0)),
        out_shape=jax.ShapeDtypeStruct((n, 1), jnp.int32),
    )(xb, x2, cb, c2)
    return out.reshape(n)


# speedup vs baseline: 1.1246x; 1.1246x over previous
import jax
import jax.numpy as jnp
from jax.experimental import pallas as pl

_BN = 2048
_CW = 128


def _half_argmin(dh, iota_cw, big):
    bn, hh = dh.shape
    val = dh[:, :_CW]
    idx = iota_cw
    for c in range(1, hh // _CW):
        chunk = dh[:, c * _CW:(c + 1) * _CW]
        lt = chunk < val
        val = jnp.where(lt, chunk, val)
        idx = jnp.where(lt, iota_cw + jnp.int32(c * _CW), idx)
    m = jnp.min(val, axis=1, keepdims=True)
    cand = jnp.where(val == m, idx, big)
    return m, jnp.min(cand, axis=1, keepdims=True)


def _assign_kernel(x_ref, x2_ref, c_ref, c2_ref, out_ref):
    x = x_ref[...]
    c = c_ref[...]
    x2 = x2_ref[...]
    c2 = c2_ref[...]
    k = c.shape[0]
    h = k // 2
    dot = jnp.dot(x, c.T, preferred_element_type=jnp.float32)
    d2 = (x2 + c2) - 2.0 * dot

    iota_cw = jax.lax.broadcasted_iota(jnp.int32, (d2.shape[0], _CW), 1)
    big = jnp.int32(h)
    ma, ia = _half_argmin(d2[:, :h], iota_cw, big)
    mb, ib = _half_argmin(d2[:, h:], iota_cw, big)

    dist_a = jnp.sqrt(jnp.maximum(ma, 0.0))
    dist_b = jnp.sqrt(jnp.maximum(mb, 0.0))
    dist_a_r = dist_a.astype(jnp.bfloat16).astype(jnp.float32)
    pick_a = dist_a_r <= dist_b
    out_ref[...] = jnp.where(pick_a, ia, ib + jnp.int32(h))


def kernel(X, centers):
    n, d = X.shape
    k, _ = centers.shape
    x2 = jnp.sum(X * X, axis=1, keepdims=True)
    c2 = jnp.sum(centers * centers, axis=1)[None, :]
    xb = X.astype(jnp.bfloat16)
    cb = centers.astype(jnp.bfloat16)
    grid = (n // _BN,)
    out = pl.pallas_call(
        _assign_kernel,
        grid=grid,
        in_specs=[
            pl.BlockSpec((_BN, d), lambda i: (i, 0)),
            pl.BlockSpec((_BN, 1), lambda i: (i, 0)),
            pl.BlockSpec((k, d), lambda i: (0, 0)),
            pl.BlockSpec((1, k), lambda i: (0, 0)),
        ],
        out_specs=pl.BlockSpec((_BN, 1), lambda i: (i, 0)),
        out_shape=jax.ShapeDtypeStruct((n, 1), jnp.int32),
    )(xb, x2, cb, c2)
    return out.reshape(n)
